# Initial kernel scaffold; baseline (speedup 1.0000x reference)
import jax
import jax.numpy as jnp
from jax.experimental import pallas as pl


def kernel(user_obs, item_obs, coef_user_obs, coef_item_obs, coef_intercept, user_index):
    raise NotImplementedError("write your pallas kernel here")


# speedup vs baseline: 1.5225x; 1.5225x over previous
import functools

import jax
import jax.numpy as jnp
from jax import lax
from jax.experimental import pallas as pl
from jax.experimental.pallas import tpu as pltpu
from jax.experimental.pallas import tpu_sc as plsc


def _sc_gather(table, idx):
    V, D = table.shape
    B = idx.shape[0]
    info = plsc.get_sparse_core_info()
    NC, NS = info.num_cores, info.num_subcores
    NW = NC * NS
    b_per_w = B // NW
    mesh = plsc.VectorSubcoreMesh(core_axis_name="c", subcore_axis_name="s")

    @functools.partial(
        pl.kernel,
        mesh=mesh,
        out_type=jax.ShapeDtypeStruct((B, D), jnp.float32),
        scratch_types=[
            pltpu.VMEM((b_per_w,), jnp.int32),
            pltpu.VMEM((b_per_w, D), jnp.float32),
            pltpu.SemaphoreType.DMA,
        ],
    )
    def k(table_hbm, idx_hbm, out_hbm, idx_v, rows_v, sem):
        wid = lax.axis_index("s") * NC + lax.axis_index("c")
        base = wid * b_per_w
        pltpu.sync_copy(idx_hbm.at[pl.ds(base, b_per_w)], idx_v)
        pltpu.async_copy(table_hbm.at[idx_v], rows_v, sem).wait()
        pltpu.sync_copy(rows_v, out_hbm.at[pl.ds(base, b_per_w)])

    return k(table, idx)


def _tc_matmul(x_u, beta_t, item_t, cvec, icpt, bm=2048):
    B, D = x_u.shape
    NIP = beta_t.shape[1]
    DI = item_t.shape[0]

    def body(x_ref, bt_ref, it_ref, cv_ref, ic_ref, o_ref):
        bias = (
            jnp.dot(cv_ref[...], it_ref[...], preferred_element_type=jnp.float32)
            + ic_ref[...]
        )
        o_ref[...] = (
            jnp.dot(x_ref[...], bt_ref[...], preferred_element_type=jnp.float32)
            + bias
        )

    return pl.pallas_call(
        body,
        grid=(B // bm,),
        in_specs=[
            pl.BlockSpec((bm, D), lambda i: (i, 0)),
            pl.BlockSpec((D, NIP), lambda i: (0, 0)),
            pl.BlockSpec((DI, NIP), lambda i: (0, 0)),
            pl.BlockSpec((1, DI), lambda i: (0, 0)),
            pl.BlockSpec((1, NIP), lambda i: (0, 0)),
        ],
        out_specs=pl.BlockSpec((bm, NIP), lambda i: (i, 0)),
        out_shape=jax.ShapeDtypeStruct((B, NIP), jnp.float32),
    )(x_u, beta_t, item_t, cvec, icpt)


def kernel(user_obs, item_obs, coef_user_obs, coef_item_obs, coef_intercept, user_index):
    V, D = user_obs.shape
    NI, DI = item_obs.shape
    NIP = 128

    beta_t = (
        jnp.zeros((D, NIP), jnp.float32).at[:, 1:NI].set(coef_user_obs.T)
    )
    icpt = jnp.zeros((1, NIP), jnp.float32).at[0, 1:NI].set(coef_intercept)
    item_t = jnp.zeros((DI, NIP), jnp.float32).at[:, :NI].set(item_obs.T)
    cvec = coef_item_obs.reshape(1, DI)

    x_u = _sc_gather(user_obs, user_index)
    util_pad = _tc_matmul(x_u, beta_t, item_t, cvec, icpt)
    return util_pad[:, :NI]
